# baseline (speedup 1.0000x reference)
import functools

import jax
import jax.numpy as jnp
import numpy as np
from jax import lax
from jax.experimental import pallas as pl
from jax.experimental.pallas import tpu as pltpu
from jax.experimental.pallas import tpu_sc as plsc

_EMBED = 64
_LANES = 16
_NC, _NS = 2, 16
_NW = _NC * _NS
_SUB = 128
_K = 4
_C = _SUB * _K


@functools.lru_cache(maxsize=None)
def _build(batch_flat: int, vocab: int, scale: float):
    assert batch_flat % (_NW * _C) == 0
    bpw = batch_flat // _NW
    nchunk = bpw // _C

    mesh = plsc.VectorSubcoreMesh(core_axis_name="c", subcore_axis_name="s")

    @functools.partial(
        pl.kernel,
        mesh=mesh,
        out_type=jax.ShapeDtypeStruct((batch_flat, _EMBED), jnp.float32),
        scratch_types=[
            pltpu.VMEM((bpw,), jnp.int32),
            pltpu.VMEM((_C, _EMBED), jnp.float32),
            pltpu.SemaphoreType.DMA,
        ],
        compiler_params=pltpu.CompilerParams(use_tc_tiling_on_sc=False),
    )
    def emb(x_hbm, tab_hbm, out_hbm, idx_v, rows_v, gsem):
        wid = lax.axis_index("s") * _NC + lax.axis_index("c")
        base_w = wid * bpw
        pltpu.sync_copy(x_hbm.at[pl.ds(base_w, bpw)], idx_v)

        def chunk(g, carry):
            off = g * _C
            cps = [
                pltpu.async_copy(
                    tab_hbm.at[idx_v.at[pl.ds(off + j * _SUB, _SUB)]],
                    rows_v.at[pl.ds(j * _SUB, _SUB)],
                    gsem,
                )
                for j in range(_K)
            ]
            for cp in cps:
                cp.wait()

            def srow(i, c2):
                for j in range(_EMBED // _LANES):
                    sl = pl.ds(j * _LANES, _LANES)
                    rows_v[i, sl] = rows_v[i, sl] * scale
                return c2

            lax.fori_loop(0, _C, srow, 0)
            pltpu.sync_copy(rows_v, out_hbm.at[pl.ds(base_w + off, _C)])
            return carry

        lax.fori_loop(0, nchunk, chunk, 0)

    return emb


def kernel(x, embedding_table):
    b, h = x.shape
    vocab, embed = embedding_table.shape
    assert embed == _EMBED
    scale = float(np.sqrt(np.float32(embed)))
    x_flat = x.reshape(b * h).astype(jnp.int32)
    emb = _build(b * h, vocab, scale)
    out = emb(x_flat, embedding_table)
    return out.reshape(b, h, _EMBED)
